# baseline (speedup 1.0000x reference)
import functools

import jax
import jax.numpy as jnp
from jax import lax
from jax.experimental import pallas as pl
from jax.experimental.pallas import tpu as pltpu
from jax.experimental.pallas import tpu_sc as plsc

BATCH = 16384
NUM_FIELDS = 26
FIELD_DIM = 40000
TOTAL_VOCAB = NUM_FIELDS * FIELD_DIM
TDIM = 16

NC, NS, LANES = 2, 16, 16
NW = NC * NS
BPW = BATCH // NW
IDX_PER_W = BPW * NUM_FIELDS
NSTAGE = 13
TSLICE = TOTAL_VOCAB // NSTAGE
CHUNK = 128
NCHUNK = BPW // CHUNK

_mesh = plsc.VectorSubcoreMesh(
    core_axis_name="c", subcore_axis_name="s", num_cores=NC, num_subcores=NS
)


@functools.partial(
    pl.kernel,
    out_type=jax.ShapeDtypeStruct((BATCH,), jnp.float32),
    mesh=_mesh,
    compiler_params=pltpu.CompilerParams(use_tc_tiling_on_sc=True),
    scratch_types=[
        pltpu.VMEM((CHUNK, NUM_FIELDS), jnp.int32),

        pltpu.VMEM((IDX_PER_W,), jnp.int32),
        pltpu.VMEM((IDX_PER_W,), jnp.int32),
        pltpu.VMEM((IDX_PER_W // 16,), jnp.int32),
        pltpu.VMEM((IDX_PER_W,), jnp.float32),
        pltpu.VMEM((BPW,), jnp.float32),
        pltpu.VMEM((2 * LANES,), jnp.float32),
        pltpu.VMEM((BPW,), jnp.float32),
        pltpu.VMEM_SHARED((TOTAL_VOCAB,), jnp.float32),
        pltpu.VMEM_SHARED((NS * IDX_PER_W // 16,), jnp.int32),
        pltpu.SemaphoreType.DMA,
    ],
)
def _fl_kernel(x_hbm, lin_hbm, tab_hbm, pv_hbm, perm_hbm, out_hbm,
               x2_v, idx_v, idxf_v, perm_v, vals_v, lin_v, pv_v, out_v,
               tab_sh, idx_sh, sem):
    c = lax.axis_index("c")
    s = lax.axis_index("s")
    wid = s * NC + c

    @pl.when(s < NSTAGE)
    def _stage():
        pltpu.sync_copy(tab_hbm.at[pl.ds(s * TSLICE, TSLICE)],
                        tab_sh.at[pl.ds(s * TSLICE, TSLICE)])

    pltpu.sync_copy(pv_hbm, pv_v)
    pltpu.sync_copy(perm_hbm, perm_v)
    pltpu.sync_copy(lin_hbm.at[pl.ds(wid * BPW, BPW)], lin_v)

    offA = lax.iota(jnp.int32, LANES) * FIELD_DIM
    offB = (lax.iota(jnp.int32, LANES) + 10) * FIELD_DIM
    for ch in range(NCHUNK):
        rowbase = wid * BPW + ch * CHUNK
        pltpu.sync_copy(x_hbm.at[pl.ds(rowbase, CHUNK), :], x2_v)

        def rowloop(i, carry, ch=ch):
            g = ch * CHUNK + i
            ia = x2_v[i, pl.ds(0, LANES)] + offA
            ib = x2_v[i, pl.ds(10, LANES)] + offB
            idx_v[pl.ds(g * NUM_FIELDS, LANES)] = ia
            idx_v[pl.ds(g * NUM_FIELDS + 10, LANES)] = ib
            return carry

        lax.fori_loop(0, CHUNK, rowloop, 0)

    NBLK = 16
    HIDX = IDX_PER_W // NBLK
    plsc.subcore_barrier()
    def tblock(h, carry):
        pltpu.sync_copy(idx_v.at[pl.ds(h * HIDX, HIDX)],
                        idx_sh.at[pl.ds(s * HIDX, HIDX)])
        pltpu.async_copy(
            idx_sh.at[pl.ds(s * HIDX, HIDX)].at[perm_v],
            idxf_v.at[pl.ds(h * HIDX, HIDX)], sem).wait()
        return carry

    lax.fori_loop(0, NBLK, tblock, 0)
    pltpu.async_copy(tab_sh.at[idxf_v], vals_v, sem).wait()

    HB = BPW // NBLK
    c0 = pv_v[pl.ds(LANES, LANES)]

    def rblock(h, carry):
        for j in range(HB // LANES):
            sl = pl.ds(h * HB + j * LANES, LANES)
            acc = lin_v[sl] + c0
            for f in range(NUM_FIELDS):
                acc = acc + vals_v[pl.ds(h * HIDX + f * HB + j * LANES, LANES)]
            out_v[sl] = acc
        return carry

    lax.fori_loop(0, NBLK, rblock, 0)
    pltpu.sync_copy(out_v, out_hbm.at[pl.ds(wid * BPW, BPW)])


LBLK = 2048


def _lin_body(t_ref, w_ref, out_ref):
    w = w_ref[0, :TDIM].reshape(1, TDIM)
    out_ref[...] = jnp.sum(t_ref[...] * w, axis=1)


def kernel(x, t, fc_table, lin_W, lin_b, bias):
    tab = fc_table.reshape(TOTAL_VOCAB)
    pv = jnp.concatenate([
        lin_W.reshape(TDIM),
        jnp.broadcast_to((lin_b + bias).reshape(1), (LANES,)),
    ]).astype(jnp.float32)
    wrow = jnp.concatenate([
        lin_W.reshape(TDIM), (lin_b + bias).reshape(1),
        jnp.zeros((128 - TDIM - 1,), jnp.float32),
    ]).reshape(1, 128)
    lin = pl.pallas_call(
        _lin_body,
        grid=(BATCH // LBLK,),
        in_specs=[
            pl.BlockSpec((LBLK, TDIM), lambda i: (i, 0)),
            pl.BlockSpec((1, 128), lambda i: (0, 0)),
        ],
        out_specs=pl.BlockSpec((LBLK,), lambda i: (i,)),
        out_shape=jax.ShapeDtypeStruct((BATCH,), jnp.float32),
    )(t, wrow)
    e = jnp.arange(IDX_PER_W // 16, dtype=jnp.int32)
    perm = (e % (BPW // 16)) * NUM_FIELDS + e // (BPW // 16)
    out = _fl_kernel(x, lin, tab, pv, perm)
    return out.reshape(BATCH, 1)


# speedup vs baseline: 1.2706x; 1.2706x over previous
import functools

import jax
import jax.numpy as jnp
from jax import lax
from jax.experimental import pallas as pl
from jax.experimental.pallas import tpu as pltpu
from jax.experimental.pallas import tpu_sc as plsc

BATCH = 16384
NUM_FIELDS = 26
FIELD_DIM = 40000
TOTAL_VOCAB = NUM_FIELDS * FIELD_DIM
TDIM = 16

NC, NS, LANES = 2, 16, 16
NW = NC * NS
BPW = BATCH // NW
IDX_PER_W = BPW * NUM_FIELDS
NSTAGE = 13
TSLICE = TOTAL_VOCAB // NSTAGE

_mesh = plsc.VectorSubcoreMesh(
    core_axis_name="c", subcore_axis_name="s", num_cores=NC, num_subcores=NS
)


@functools.partial(
    pl.kernel,
    out_type=jax.ShapeDtypeStruct((BATCH,), jnp.float32),
    mesh=_mesh,
    compiler_params=pltpu.CompilerParams(use_tc_tiling_on_sc=True),
    scratch_types=[
        pltpu.VMEM((IDX_PER_W,), jnp.int32),
        pltpu.VMEM((IDX_PER_W,), jnp.float32),
        pltpu.VMEM((TDIM * BPW,), jnp.float32),
        pltpu.VMEM((TDIM * LANES + LANES,), jnp.float32),
        pltpu.VMEM((BPW,), jnp.float32),
        pltpu.VMEM_SHARED((TOTAL_VOCAB,), jnp.float32),
        pltpu.SemaphoreType.DMA,
    ],
)
def _fl_kernel(xw_hbm, tb_hbm, tab_hbm, pv_hbm, out_hbm,
               idx_v, vals_v, tb_v, pv_v, out_v, tab_sh, sem):
    c = lax.axis_index("c")
    s = lax.axis_index("s")
    wid = s * NC + c

    @pl.when(s < NSTAGE)
    def _stage():
        pltpu.sync_copy(tab_hbm.at[pl.ds(s * TSLICE, TSLICE)],
                        tab_sh.at[pl.ds(s * TSLICE, TSLICE)])

    pltpu.sync_copy(xw_hbm.at[pl.ds(wid * IDX_PER_W, IDX_PER_W)], idx_v)
    pltpu.sync_copy(tb_hbm.at[pl.ds(wid * TDIM * BPW, TDIM * BPW)], tb_v)
    pltpu.sync_copy(pv_hbm, pv_v)

    plsc.subcore_barrier()

    pltpu.async_copy(tab_sh.at[idx_v], vals_v, sem).wait()

    c0 = pv_v[pl.ds(TDIM * LANES, LANES)]
    for j in range(BPW // LANES):
        acc = c0
        for f in range(NUM_FIELDS):
            acc = acc + vals_v[pl.ds(f * BPW + j * LANES, LANES)]
        for k in range(TDIM):
            acc = acc + pv_v[pl.ds(k * LANES, LANES)] * tb_v[pl.ds(k * BPW + j * LANES, LANES)]
        out_v[pl.ds(j * LANES, LANES)] = acc
    pltpu.sync_copy(out_v, out_hbm.at[pl.ds(wid * BPW, BPW)])


def kernel(x, t, fc_table, lin_W, lin_b, bias):
    offsets = jnp.arange(NUM_FIELDS, dtype=x.dtype) * FIELD_DIM
    xi = x + offsets[None, :]
    xw = xi.reshape(NW, BPW, NUM_FIELDS).transpose(0, 2, 1).reshape(NW * IDX_PER_W)
    tb = t.reshape(NW, BPW, TDIM).transpose(0, 2, 1).reshape(NW * TDIM * BPW)
    tab = fc_table.reshape(TOTAL_VOCAB)
    pv = jnp.concatenate([
        jnp.repeat(lin_W.reshape(TDIM), LANES),
        jnp.broadcast_to((lin_b + bias).reshape(1), (LANES,)),
    ]).astype(jnp.float32)
    out = _fl_kernel(xw, tb, tab, pv)
    return out.reshape(BATCH, 1)
